# baseline (speedup 1.0000x reference)
import functools

import jax
import jax.numpy as jnp
from jax import lax
from jax.experimental import pallas as pl
from jax.experimental.pallas import tpu as pltpu
from jax.experimental.pallas import tpu_sc as plsc

T, B, D = 8192, 4, 1024
N = T * B
NC, NS, L = 2, 16, 16
NW = NC * NS
ROWS_PER_W = N // NW
K = 16
NCHUNK = ROWS_PER_W // K


def _cumsum_body(f_ref, idx_ref):
    f = f_ref[...]
    t = lax.broadcasted_iota(jnp.int32, (T, B), 0)
    c = jnp.where(t == 0, 0, f.astype(jnp.int32))
    k = 1
    while k < T:
        z = jnp.zeros((k, B), jnp.int32)
        c = c + jnp.concatenate([z, c[:-k, :]], axis=0)
        k *= 2
    idx_ref[...] = c


def _cumsum(flags):
    return pl.pallas_call(
        _cumsum_body,
        out_shape=jax.ShapeDtypeStruct((T, B), jnp.int32),
    )(flags)


def _sc_body(idx_hbm, x_hbm, tbl_hbm, out_hbm, idx_v, pe_v, x_v, sem):
    c = lax.axis_index("c")
    s = lax.axis_index("s")
    wid = s * NC + c
    base = wid * ROWS_PER_W
    pltpu.sync_copy(idx_hbm.at[pl.ds(base, ROWS_PER_W)], idx_v)

    def chunk(g, carry):
        r0 = base + g * K
        gat = pltpu.async_copy(tbl_hbm.at[idx_v.at[pl.ds(g * K, K)]], pe_v, sem)
        pltpu.sync_copy(x_hbm.at[pl.ds(r0, K)], x_v)
        gat.wait()

        def add_row(r, carry2):
            def add_col(j, carry3):
                sl = pl.ds(j * L, L)
                x_v[r, sl] = x_v[r, sl] + pe_v[r, sl]
                return 0
            return lax.fori_loop(0, D // L, add_col, 0)

        lax.fori_loop(0, K, add_row, 0)
        pltpu.sync_copy(x_v, out_hbm.at[pl.ds(r0, K)])
        return 0

    lax.fori_loop(0, NCHUNK, chunk, 0)


def _sc_gather_add(idx, x, tbl):
    mesh = plsc.VectorSubcoreMesh(core_axis_name="c", subcore_axis_name="s")
    fn = functools.partial(
        pl.kernel,
        mesh=mesh,
        out_type=jax.ShapeDtypeStruct((N, D), jnp.float32),
        scratch_types=[
            pltpu.VMEM((ROWS_PER_W,), jnp.int32),
            pltpu.VMEM((K, D), jnp.float32),
            pltpu.VMEM((K, D), jnp.float32),
            pltpu.SemaphoreType.DMA,
        ],
    )(_sc_body)
    return fn(idx, x, tbl)


def kernel(x_original, x_projected_to_d_model, pe_table):
    flags = x_original[:, :, -1]
    idx = _cumsum(flags).reshape(N)
    x = x_projected_to_d_model.reshape(N, D)
    out = _sc_gather_add(idx, x, pe_table)
    return out.reshape(T, B, D)


# speedup vs baseline: 1.0366x; 1.0366x over previous
import functools

import jax
import jax.numpy as jnp
from jax import lax
from jax.experimental import pallas as pl
from jax.experimental.pallas import tpu as pltpu
from jax.experimental.pallas import tpu_sc as plsc

T, B, D = 8192, 4, 1024
N = T * B
NC, NS, L = 2, 16, 16
NW = NC * NS
ROWS_PER_W = N // NW
K = 16
NCHUNK = ROWS_PER_W // K
NBUF = 2


def _cumsum_body(f_ref, idx_ref):
    f = f_ref[...]
    t = lax.broadcasted_iota(jnp.int32, (T, B), 0)
    c = jnp.where(t == 0, 0, f.astype(jnp.int32))
    k = 1
    while k < T:
        z = jnp.zeros((k, B), jnp.int32)
        c = c + jnp.concatenate([z, c[:-k, :]], axis=0)
        k *= 2
    idx_ref[...] = c


def _cumsum(flags):
    return pl.pallas_call(
        _cumsum_body,
        out_shape=jax.ShapeDtypeStruct((T, B), jnp.int32),
    )(flags)


def _sc_body(idx_hbm, x_hbm, tbl_hbm, out_hbm, idx_v,
             pe0, pe1, x0, x1, gsem0, gsem1, xsem0, xsem1, osem0, osem1):
    c = lax.axis_index("c")
    s = lax.axis_index("s")
    wid = s * NC + c
    base = wid * ROWS_PER_W
    pe = (pe0, pe1)
    xb = (x0, x1)
    gsem = (gsem0, gsem1)
    xsem = (xsem0, xsem1)
    osem = (osem0, osem1)

    pltpu.sync_copy(idx_hbm.at[pl.ds(base, ROWS_PER_W)], idx_v)

    def issue_in(g, b):
        r0 = base + g * K
        pltpu.async_copy(tbl_hbm.at[idx_v.at[pl.ds(g * K, K)]], pe[b], gsem[b])
        pltpu.async_copy(x_hbm.at[pl.ds(r0, K)], xb[b], xsem[b])

    for b in range(NBUF):
        issue_in(b, b)

    def pair(p, _):
        for b in range(NBUF):
            g = p * NBUF + b
            pltpu.make_async_copy(x_hbm.at[pl.ds(base, K)], pe[b], gsem[b]).wait()
            pltpu.make_async_copy(x_hbm.at[pl.ds(base, K)], xb[b], xsem[b]).wait()
            for r in range(K):
                for j in range(D // L):
                    sl = pl.ds(j * L, L)
                    plsc.addupdate(xb[b].at[r, sl], pe[b][r, sl])
            r0 = base + g * K
            pltpu.async_copy(xb[b], out_hbm.at[pl.ds(r0, K)], osem[b])

            @pl.when(g + NBUF < NCHUNK)
            def _():
                pltpu.make_async_copy(xb[b], out_hbm.at[pl.ds(base, K)], osem[b]).wait()
                issue_in(g + NBUF, b)
        return 0

    lax.fori_loop(0, NCHUNK // NBUF, pair, 0)
    for b in range(NBUF):
        pltpu.make_async_copy(xb[b], out_hbm.at[pl.ds(base, K)], osem[b]).wait()


def _sc_gather_add(idx, x, tbl):
    mesh = plsc.VectorSubcoreMesh(core_axis_name="c", subcore_axis_name="s")
    fn = functools.partial(
        pl.kernel,
        mesh=mesh,
        out_type=jax.ShapeDtypeStruct((N, D), jnp.float32),
        scratch_types=[
            pltpu.VMEM((ROWS_PER_W,), jnp.int32),
            pltpu.VMEM((K, D), jnp.float32),
            pltpu.VMEM((K, D), jnp.float32),
            pltpu.VMEM((K, D), jnp.float32),
            pltpu.VMEM((K, D), jnp.float32),
            pltpu.SemaphoreType.DMA,
            pltpu.SemaphoreType.DMA,
            pltpu.SemaphoreType.DMA,
            pltpu.SemaphoreType.DMA,
            pltpu.SemaphoreType.DMA,
            pltpu.SemaphoreType.DMA,
        ],
    )(_sc_body)
    return fn(idx, x, tbl)


def kernel(x_original, x_projected_to_d_model, pe_table):
    flags = x_original[:, :, -1]
    idx = _cumsum(flags).reshape(N)
    x = x_projected_to_d_model.reshape(N, D)
    out = _sc_gather_add(idx, x, pe_table)
    return out.reshape(T, B, D)


# speedup vs baseline: 2.5937x; 2.5022x over previous
import functools

import jax
import jax.numpy as jnp
from jax import lax
from jax.experimental import pallas as pl
from jax.experimental.pallas import tpu as pltpu
from jax.experimental.pallas import tpu_sc as plsc

T, B, D = 8192, 4, 1024
N = T * B
NC, NS, L = 2, 16, 16
NW = NC * NS
T_PER_W = T // NW
KT = 4
K = KT * B
NCHUNK = T_PER_W // KT
NBUF = 2


def _cumsum_body(f_ref, idx_ref):
    f = f_ref[...]
    t = lax.broadcasted_iota(jnp.int32, (T, B), 0)
    c = jnp.where(t == 0, 0, f.astype(jnp.int32))
    k = 1
    while k < T:
        z = jnp.zeros((k, B), jnp.int32)
        c = c + jnp.concatenate([z, c[:-k, :]], axis=0)
        k *= 2
    idx_ref[...] = c


def _cumsum(flags):
    return pl.pallas_call(
        _cumsum_body,
        out_shape=jax.ShapeDtypeStruct((T, B), jnp.int32),
    )(flags)


def _sc_body(idx_hbm, x_hbm, tbl_hbm, out_hbm, idx_v,
             pe0, pe1, x0, x1, gsem0, gsem1, xsem0, xsem1, osem0, osem1):
    cc = lax.axis_index("c")
    ss = lax.axis_index("s")
    wid = ss * NC + cc
    t0 = wid * T_PER_W
    base = t0 * B
    pe = (pe0, pe1)
    xb = (x0, x1)
    gsem = (gsem0, gsem1)
    xsem = (xsem0, xsem1)
    osem = (osem0, osem1)

    pltpu.sync_copy(idx_hbm.at[pl.ds(base, T_PER_W * B)], idx_v)

    def issue_in(g, b):
        pltpu.async_copy(tbl_hbm.at[idx_v.at[pl.ds(g * K, K)]], pe[b], gsem[b])
        pltpu.async_copy(x_hbm.at[pl.ds(t0 + g * KT, KT)], xb[b], xsem[b])

    issue_in(0, 0)

    def pair(p, _):
        for b in range(NBUF):
            o = 1 - b
            g = p * NBUF + b

            @pl.when(g >= 1)
            def _():
                pltpu.make_async_copy(xb[o], out_hbm.at[pl.ds(t0, KT)], osem[o]).wait()

            @pl.when(g + 1 < NCHUNK)
            def _():
                issue_in(g + 1, o)

            pltpu.make_async_copy(x_hbm.at[pl.ds(t0, KT)], pe[b], gsem[b]).wait()
            pltpu.make_async_copy(x_hbm.at[pl.ds(t0, KT)], xb[b], xsem[b]).wait()
            def add_row(r, _):
                t = r // B
                bb = lax.rem(r, B)

                def add_col(jo, __):
                    for ji in range(8):
                        sl = pl.ds(jo * (8 * L) + ji * L, L)
                        plsc.addupdate(xb[b].at[t, bb, sl], pe[b][r, sl])
                    return 0

                return lax.fori_loop(0, D // (8 * L), add_col, 0)

            lax.fori_loop(0, K, add_row, 0)
            pltpu.async_copy(xb[b], out_hbm.at[pl.ds(t0 + g * KT, KT)], osem[b])
        return 0

    lax.fori_loop(0, NCHUNK // NBUF, pair, 0)
    bl = (NCHUNK - 1) % NBUF
    pltpu.make_async_copy(xb[bl], out_hbm.at[pl.ds(t0, KT)], osem[bl]).wait()


def _sc_gather_add(idx, x, tbl):
    mesh = plsc.VectorSubcoreMesh(core_axis_name="c", subcore_axis_name="s")
    fn = functools.partial(
        pl.kernel,
        mesh=mesh,
        out_type=jax.ShapeDtypeStruct((T, B, D), jnp.float32),
        scratch_types=[
            pltpu.VMEM((T_PER_W * B,), jnp.int32),
            pltpu.VMEM((K, D), jnp.float32),
            pltpu.VMEM((K, D), jnp.float32),
            pltpu.VMEM((KT, B, D), jnp.float32),
            pltpu.VMEM((KT, B, D), jnp.float32),
            pltpu.SemaphoreType.DMA,
            pltpu.SemaphoreType.DMA,
            pltpu.SemaphoreType.DMA,
            pltpu.SemaphoreType.DMA,
            pltpu.SemaphoreType.DMA,
            pltpu.SemaphoreType.DMA,
        ],
    )(_sc_body)
    return fn(idx, x, tbl)


def kernel(x_original, x_projected_to_d_model, pe_table):
    flags = x_original[:, :, -1]
    idx = _cumsum(flags).reshape(N)
    return _sc_gather_add(idx, x_projected_to_d_model, pe_table)
